# baseline (device time: 147633 ns/iter reference)
import jax
import jax.numpy as jnp
from jax import lax
from jax.experimental import pallas as pl
from jax.experimental.pallas import tpu as pltpu

B, H, D, BS = 32, 16, 128, 32
NSLOTS = 256
P = 256
NK = P * BS
PG = 64
SCALE = D ** -0.5


def kernel(Q, K, V, bt, lens):
    lens2 = lens.reshape(B, 1)
    Qr = Q.reshape(B, H * D)
    Kr = K.reshape(NK, H * D)
    Vr = V.reshape(NK, H * D)

    def body(q_ref, k_ref, v_ref, bt_ref, lens_ref, out_ref,
             w_keys, acc_o, acc_l, recv_o, recv_l, send_sems, recv_sems):
        h = pl.program_id(0)
        my_x = lax.axis_index("x")
        my_y = lax.axis_index("y")

        @pl.when(h == 0)
        def _init():
            acc_o[...] = jnp.zeros_like(acc_o)
            acc_l[...] = jnp.zeros_like(acc_l)
            barrier = pltpu.get_barrier_semaphore()
            pl.semaphore_signal(
                barrier, inc=1,
                device_id=(my_x, 1 - my_y),
                device_id_type=pl.DeviceIdType.MESH,
            )
            pl.semaphore_wait(barrier, 1)

            slot = lax.broadcasted_iota(jnp.int32, (1, 1, NSLOTS), 2)
            valid = slot < lens_ref[...][:, :, None]
            bt3 = bt_ref[...][:, None, :]
            parts = []
            for g in range(P // PG):
                page_ids = (
                    my_y * P + g * PG
                    + lax.broadcasted_iota(jnp.int32, (1, PG, 1), 1)
                )
                hit = (bt3 == page_ids) & valid
                parts.append(jnp.sum(jnp.where(hit, 1.0, 0.0), axis=2))
            w_page = jnp.concatenate(parts, axis=1)
            w_keys[...] = jnp.broadcast_to(
                w_page[:, :, None], (B, P, BS)
            ).reshape(B, NK)

        qh = q_ref[...]
        kh = k_ref[...]
        vh = v_ref[...]
        s = lax.dot_general(
            qh, kh, (((1,), (1,)), ((), ())),
            preferred_element_type=jnp.float32,
        ) * SCALE
        p = jnp.exp(s) * w_keys[...]
        acc_l[h, :] = jnp.sum(p, axis=1)
        acc_o[h, :, :] = lax.dot_general(
            p, vh, (((1,), (0,)), ((), ())),
            preferred_element_type=jnp.float32,
        )

        @pl.when(h == H - 1)
        def _exchange():
            nbr = (my_x, 1 - my_y)
            rdma_o = pltpu.make_async_remote_copy(
                src_ref=acc_o, dst_ref=recv_o,
                send_sem=send_sems.at[0], recv_sem=recv_sems.at[0],
                device_id=nbr, device_id_type=pl.DeviceIdType.MESH,
            )
            rdma_l = pltpu.make_async_remote_copy(
                src_ref=acc_l, dst_ref=recv_l,
                send_sem=send_sems.at[1], recv_sem=recv_sems.at[1],
                device_id=nbr, device_id_type=pl.DeviceIdType.MESH,
            )
            rdma_o.start()
            rdma_l.start()
            rdma_o.wait()
            rdma_l.wait()
            l_tot = acc_l[...] + recv_l[...]
            o_tot = acc_o[...] + recv_o[...]
            o = o_tot / l_tot[:, :, None]
            out_ref[...] = jnp.transpose(o, (1, 0, 2)).reshape(B, H * D)

    out2 = pl.pallas_call(
        body,
        grid=(H,),
        in_specs=[
            pl.BlockSpec((B, D), lambda h: (0, h)),
            pl.BlockSpec((NK, D), lambda h: (0, h)),
            pl.BlockSpec((NK, D), lambda h: (0, h)),
            pl.BlockSpec((B, NSLOTS), lambda h: (0, 0)),
            pl.BlockSpec((B, 1), lambda h: (0, 0)),
        ],
        out_specs=pl.BlockSpec((B, H * D), lambda h: (0, 0)),
        out_shape=jax.ShapeDtypeStruct((B, H * D), jnp.float32),
        scratch_shapes=[
            pltpu.VMEM((B, NK), jnp.float32),
            pltpu.VMEM((H, B, D), jnp.float32),
            pltpu.VMEM((H, B), jnp.float32),
            pltpu.VMEM((H, B, D), jnp.float32),
            pltpu.VMEM((H, B), jnp.float32),
            pltpu.SemaphoreType.DMA((2,)),
            pltpu.SemaphoreType.DMA((2,)),
        ],
        compiler_params=pltpu.CompilerParams(
            collective_id=0,
            dimension_semantics=("arbitrary",),
        ),
    )(Qr, Kr, Vr, bt, lens2)
    return out2.reshape(B, 1, H, D)


# device time: 52865 ns/iter; 2.7926x vs baseline; 2.7926x over previous
import jax
import jax.numpy as jnp
from jax import lax
from jax.experimental import pallas as pl
from jax.experimental.pallas import tpu as pltpu

B, H, D, BS = 32, 16, 128, 32
NSLOTS = 256
P = 256
NK = P * BS
PG = 64
SCALE = D ** -0.5


def kernel(Q, K, V, bt, lens):
    lens2 = lens.reshape(B, 1)
    Qr = Q.reshape(B, H * D)

    def body(q_ref, k_hbm, v_hbm, bt_ref, lens_ref, out_ref,
             w_keys, acc_o, acc_l, recv_o, recv_l,
             kbuf, vbuf, copy_sems, send_sems, recv_sems):
        h = pl.program_id(0)
        my_x = lax.axis_index("x")
        my_y = lax.axis_index("y")

        def start_copies(hh, slot):
            pltpu.make_async_copy(
                k_hbm.at[:, :, hh, :], kbuf.at[slot], copy_sems.at[0, slot]
            ).start()
            pltpu.make_async_copy(
                v_hbm.at[:, :, hh, :], vbuf.at[slot], copy_sems.at[1, slot]
            ).start()

        @pl.when(h == 0)
        def _init():
            start_copies(0, 0)
            acc_o[...] = jnp.zeros_like(acc_o)
            acc_l[...] = jnp.zeros_like(acc_l)
            barrier = pltpu.get_barrier_semaphore()
            pl.semaphore_signal(
                barrier, inc=1,
                device_id=(my_x, 1 - my_y),
                device_id_type=pl.DeviceIdType.MESH,
            )
            pl.semaphore_wait(barrier, 1)

            slot = lax.broadcasted_iota(jnp.int32, (1, 1, NSLOTS), 2)
            valid = slot < lens_ref[...][:, :, None]
            bt3 = bt_ref[...][:, None, :]
            parts = []
            for g in range(P // PG):
                page_ids = (
                    my_y * P + g * PG
                    + lax.broadcasted_iota(jnp.int32, (1, PG, 1), 1)
                )
                hit = (bt3 == page_ids) & valid
                parts.append(jnp.sum(jnp.where(hit, 1.0, 0.0), axis=2))
            w_page = jnp.concatenate(parts, axis=1)
            w_keys[...] = jnp.broadcast_to(
                w_page[:, :, None], (B, P, BS)
            ).reshape(B, NK)

        @pl.when(h + 1 < H)
        def _prefetch():
            start_copies(h + 1, (h + 1) % 2)

        pltpu.make_async_copy(
            k_hbm.at[:, :, h, :], kbuf.at[h % 2], copy_sems.at[0, h % 2]
        ).wait()
        pltpu.make_async_copy(
            v_hbm.at[:, :, h, :], vbuf.at[h % 2], copy_sems.at[1, h % 2]
        ).wait()

        qh = q_ref[...]
        kh = kbuf[h % 2].reshape(NK, D)
        vh = vbuf[h % 2].reshape(NK, D)
        s = lax.dot_general(
            qh, kh, (((1,), (1,)), ((), ())),
            preferred_element_type=jnp.float32,
        ) * SCALE
        p = jnp.exp(s) * w_keys[...]
        acc_l[h, :] = jnp.sum(p, axis=1)
        acc_o[h, :, :] = lax.dot_general(
            p, vh, (((1,), (0,)), ((), ())),
            preferred_element_type=jnp.float32,
        )

        @pl.when(h == H - 1)
        def _exchange():
            nbr = (my_x, 1 - my_y)
            rdma_o = pltpu.make_async_remote_copy(
                src_ref=acc_o, dst_ref=recv_o,
                send_sem=send_sems.at[0], recv_sem=recv_sems.at[0],
                device_id=nbr, device_id_type=pl.DeviceIdType.MESH,
            )
            rdma_l = pltpu.make_async_remote_copy(
                src_ref=acc_l, dst_ref=recv_l,
                send_sem=send_sems.at[1], recv_sem=recv_sems.at[1],
                device_id=nbr, device_id_type=pl.DeviceIdType.MESH,
            )
            rdma_o.start()
            rdma_l.start()
            rdma_o.wait()
            rdma_l.wait()
            l_tot = acc_l[...] + recv_l[...]
            o_tot = acc_o[...] + recv_o[...]
            o = o_tot / l_tot[:, :, None]
            out_ref[...] = jnp.transpose(o, (1, 0, 2)).reshape(B, H * D)

    out2 = pl.pallas_call(
        body,
        grid=(H,),
        in_specs=[
            pl.BlockSpec((B, D), lambda h: (0, h)),
            pl.BlockSpec(memory_space=pltpu.MemorySpace.HBM),
            pl.BlockSpec(memory_space=pltpu.MemorySpace.HBM),
            pl.BlockSpec((B, NSLOTS), lambda h: (0, 0)),
            pl.BlockSpec((B, 1), lambda h: (0, 0)),
        ],
        out_specs=pl.BlockSpec((B, H * D), lambda h: (0, 0)),
        out_shape=jax.ShapeDtypeStruct((B, H * D), jnp.float32),
        scratch_shapes=[
            pltpu.VMEM((B, NK), jnp.float32),
            pltpu.VMEM((H, B, D), jnp.float32),
            pltpu.VMEM((H, B), jnp.float32),
            pltpu.VMEM((H, B, D), jnp.float32),
            pltpu.VMEM((H, B), jnp.float32),
            pltpu.VMEM((2, P, BS, D), jnp.float32),
            pltpu.VMEM((2, P, BS, D), jnp.float32),
            pltpu.SemaphoreType.DMA((2, 2)),
            pltpu.SemaphoreType.DMA((2,)),
            pltpu.SemaphoreType.DMA((2,)),
        ],
        compiler_params=pltpu.CompilerParams(
            collective_id=0,
            dimension_semantics=("arbitrary",),
        ),
    )(Qr, K, V, bt, lens2)
    return out2.reshape(B, 1, H, D)


# device time: 52614 ns/iter; 2.8060x vs baseline; 1.0048x over previous
import jax
import jax.numpy as jnp
from jax import lax
from jax.experimental import pallas as pl
from jax.experimental.pallas import tpu as pltpu

B, H, D, BS = 32, 16, 128, 32
NSLOTS = 256
P = 256
NK = P * BS
PG = 64
SCALE = D ** -0.5


def kernel(Q, K, V, bt, lens):
    lens2 = lens.reshape(B, 1)
    Qr = Q.reshape(B, H * D)

    def body(q_ref, k_hbm, v_hbm, bt_ref, lens_ref, out_ref,
             w_keys, acc_o, acc_l, recv_o, recv_l,
             kbuf, vbuf, copy_sems, send_sems, recv_sems):
        h = pl.program_id(0)
        my_x = lax.axis_index("x")
        my_y = lax.axis_index("y")

        def start_copies(hh, slot):
            for half in range(2):
                sl = pl.ds(half * (P // 2), P // 2)
                pltpu.make_async_copy(
                    k_hbm.at[sl, :, hh, :], kbuf.at[slot, sl],
                    copy_sems.at[0, slot, half],
                ).start()
                pltpu.make_async_copy(
                    v_hbm.at[sl, :, hh, :], vbuf.at[slot, sl],
                    copy_sems.at[1, slot, half],
                ).start()

        def wait_copies(hh, slot):
            for half in range(2):
                sl = pl.ds(half * (P // 2), P // 2)
                pltpu.make_async_copy(
                    k_hbm.at[sl, :, hh, :], kbuf.at[slot, sl],
                    copy_sems.at[0, slot, half],
                ).wait()
                pltpu.make_async_copy(
                    v_hbm.at[sl, :, hh, :], vbuf.at[slot, sl],
                    copy_sems.at[1, slot, half],
                ).wait()

        @pl.when(h == 0)
        def _init():
            start_copies(0, 0)
            acc_o[...] = jnp.zeros_like(acc_o)
            acc_l[...] = jnp.zeros_like(acc_l)
            barrier = pltpu.get_barrier_semaphore()
            pl.semaphore_signal(
                barrier, inc=1,
                device_id=(my_x, 1 - my_y),
                device_id_type=pl.DeviceIdType.MESH,
            )
            pl.semaphore_wait(barrier, 1)

            slot = lax.broadcasted_iota(jnp.int32, (1, 1, NSLOTS), 2)
            valid = slot < lens_ref[...][:, :, None]
            bt3 = bt_ref[...][:, None, :]
            parts = []
            for g in range(P // PG):
                page_ids = (
                    my_y * P + g * PG
                    + lax.broadcasted_iota(jnp.int32, (1, PG, 1), 1)
                )
                hit = (bt3 == page_ids) & valid
                parts.append(jnp.sum(jnp.where(hit, 1.0, 0.0), axis=2))
            w_page = jnp.concatenate(parts, axis=1)
            w_keys[...] = jnp.broadcast_to(
                w_page[:, :, None], (B, P, BS)
            ).reshape(B, NK)

        @pl.when(h + 1 < H)
        def _prefetch():
            start_copies(h + 1, (h + 1) % 2)

        wait_copies(h, h % 2)

        qh = q_ref[...] * SCALE
        kh = kbuf[h % 2].reshape(NK, D)
        vh = vbuf[h % 2].reshape(NK, D)
        s = lax.dot_general(
            qh, kh, (((1,), (1,)), ((), ())),
            preferred_element_type=jnp.float32,
        )
        p = jnp.exp(s) * w_keys[...]
        acc_l[h, :] = jnp.sum(p, axis=1)
        acc_o[h, :, :] = lax.dot_general(
            p, vh, (((1,), (0,)), ((), ())),
            preferred_element_type=jnp.float32,
        )

        @pl.when(h == H - 1)
        def _exchange():
            nbr = (my_x, 1 - my_y)
            rdma_o = pltpu.make_async_remote_copy(
                src_ref=acc_o, dst_ref=recv_o,
                send_sem=send_sems.at[0], recv_sem=recv_sems.at[0],
                device_id=nbr, device_id_type=pl.DeviceIdType.MESH,
            )
            rdma_l = pltpu.make_async_remote_copy(
                src_ref=acc_l, dst_ref=recv_l,
                send_sem=send_sems.at[1], recv_sem=recv_sems.at[1],
                device_id=nbr, device_id_type=pl.DeviceIdType.MESH,
            )
            rdma_o.start()
            rdma_l.start()
            rdma_o.wait()
            rdma_l.wait()
            l_tot = acc_l[...] + recv_l[...]
            o_tot = acc_o[...] + recv_o[...]
            o = o_tot / l_tot[:, :, None]
            out_ref[...] = jnp.transpose(o, (1, 0, 2)).reshape(B, H * D)

    out2 = pl.pallas_call(
        body,
        grid=(H,),
        in_specs=[
            pl.BlockSpec((B, D), lambda h: (0, h)),
            pl.BlockSpec(memory_space=pltpu.MemorySpace.HBM),
            pl.BlockSpec(memory_space=pltpu.MemorySpace.HBM),
            pl.BlockSpec((B, NSLOTS), lambda h: (0, 0)),
            pl.BlockSpec((B, 1), lambda h: (0, 0)),
        ],
        out_specs=pl.BlockSpec((B, H * D), lambda h: (0, 0)),
        out_shape=jax.ShapeDtypeStruct((B, H * D), jnp.float32),
        scratch_shapes=[
            pltpu.VMEM((B, NK), jnp.float32),
            pltpu.VMEM((H, B, D), jnp.float32),
            pltpu.VMEM((H, B), jnp.float32),
            pltpu.VMEM((H, B, D), jnp.float32),
            pltpu.VMEM((H, B), jnp.float32),
            pltpu.VMEM((2, P, BS, D), jnp.float32),
            pltpu.VMEM((2, P, BS, D), jnp.float32),
            pltpu.SemaphoreType.DMA((2, 2, 2)),
            pltpu.SemaphoreType.DMA((2,)),
            pltpu.SemaphoreType.DMA((2,)),
        ],
        compiler_params=pltpu.CompilerParams(
            collective_id=0,
            dimension_semantics=("arbitrary",),
        ),
    )(Qr, K, V, bt, lens2)
    return out2.reshape(B, 1, H, D)


# device time: 35759 ns/iter; 4.1286x vs baseline; 1.4713x over previous
import jax
import jax.numpy as jnp
from jax import lax
from jax.experimental import pallas as pl
from jax.experimental.pallas import tpu as pltpu

B, H, D, BS = 32, 16, 128, 32
HG = H // 2
NSLOTS = 256
P = 256
NK = P * BS
PG = 64
SCALE = D ** -0.5


def kernel(Q, K, V, bt, lens):
    lens2 = lens.reshape(B, 1)

    def body(q_hbm, k_hbm, v_hbm, bt_ref, lens_ref, out_ref,
             w_keys, acc_o, acc_l, recv_o, recv_l, xsend, xrecv,
             qbuf, kbuf, vbuf, copy_sems, send_sems, recv_sems):
        h = pl.program_id(0)
        my_x = lax.axis_index("x")
        my_y = lax.axis_index("y")
        hg = my_x * HG + h

        def copy_ops(hh, slot):
            ops = [
                pltpu.make_async_copy(
                    q_hbm.at[:, 0, my_x * HG + hh, :], qbuf.at[slot],
                    copy_sems.at[2, slot, 0],
                )
            ]
            for half in range(2):
                sl = pl.ds(half * (P // 2), P // 2)
                ops.append(pltpu.make_async_copy(
                    k_hbm.at[sl, :, my_x * HG + hh, :], kbuf.at[slot, sl],
                    copy_sems.at[0, slot, half],
                ))
                ops.append(pltpu.make_async_copy(
                    v_hbm.at[sl, :, my_x * HG + hh, :], vbuf.at[slot, sl],
                    copy_sems.at[1, slot, half],
                ))
            return ops

        @pl.when(h == 0)
        def _init():
            for op in copy_ops(0, 0):
                op.start()
            acc_o[...] = jnp.zeros_like(acc_o)
            acc_l[...] = jnp.zeros_like(acc_l)
            barrier = pltpu.get_barrier_semaphore()
            for nbr in ((my_x, 1 - my_y), (1 - my_x, my_y)):
                pl.semaphore_signal(
                    barrier, inc=1,
                    device_id=nbr, device_id_type=pl.DeviceIdType.MESH,
                )
            pl.semaphore_wait(barrier, 2)

            slot = lax.broadcasted_iota(jnp.int32, (1, 1, NSLOTS), 2)
            valid = slot < lens_ref[...][:, :, None]
            bt3 = bt_ref[...][:, None, :]
            parts = []
            for g in range(P // PG):
                page_ids = (
                    my_y * P + g * PG
                    + lax.broadcasted_iota(jnp.int32, (1, PG, 1), 1)
                )
                hit = (bt3 == page_ids) & valid
                parts.append(jnp.sum(jnp.where(hit, 1.0, 0.0), axis=2))
            w_page = jnp.concatenate(parts, axis=1)
            w_keys[...] = jnp.broadcast_to(
                w_page[:, :, None], (B, P, BS)
            ).reshape(B, NK)

        @pl.when(h + 1 < HG)
        def _prefetch():
            for op in copy_ops(h + 1, (h + 1) % 2):
                op.start()

        for op in copy_ops(h, h % 2):
            op.wait()

        qh = qbuf[h % 2] * SCALE
        kh = kbuf[h % 2].reshape(NK, D)
        vh = vbuf[h % 2].reshape(NK, D)
        s = lax.dot_general(
            qh, kh, (((1,), (1,)), ((), ())),
            preferred_element_type=jnp.float32,
        )
        p = jnp.exp(s) * w_keys[...]
        acc_l[h, :] = jnp.sum(p, axis=1)
        acc_o[h, :, :] = lax.dot_general(
            p, vh, (((1,), (0,)), ((), ())),
            preferred_element_type=jnp.float32,
        )

        @pl.when(h == HG - 1)
        def _exchange():
            ynbr = (my_x, 1 - my_y)
            rdma_o = pltpu.make_async_remote_copy(
                src_ref=acc_o, dst_ref=recv_o,
                send_sem=send_sems.at[0], recv_sem=recv_sems.at[0],
                device_id=ynbr, device_id_type=pl.DeviceIdType.MESH,
            )
            rdma_l = pltpu.make_async_remote_copy(
                src_ref=acc_l, dst_ref=recv_l,
                send_sem=send_sems.at[1], recv_sem=recv_sems.at[1],
                device_id=ynbr, device_id_type=pl.DeviceIdType.MESH,
            )
            rdma_o.start()
            rdma_l.start()
            rdma_o.wait()
            rdma_l.wait()
            l_tot = acc_l[...] + recv_l[...]
            o_tot = acc_o[...] + recv_o[...]
            xsend[...] = o_tot / l_tot[:, :, None]

            xnbr = (1 - my_x, my_y)
            rdma_x = pltpu.make_async_remote_copy(
                src_ref=xsend, dst_ref=xrecv,
                send_sem=send_sems.at[2], recv_sem=recv_sems.at[2],
                device_id=xnbr, device_id_type=pl.DeviceIdType.MESH,
            )
            rdma_x.start()
            rdma_x.wait()

            mine = xsend[...]
            theirs = xrecv[...]
            is_x0 = (my_x == 0)
            g0 = jnp.where(is_x0, mine, theirs)
            g1 = jnp.where(is_x0, theirs, mine)
            o_all = jnp.concatenate([g0, g1], axis=0)
            out_ref[...] = jnp.transpose(o_all, (1, 0, 2)).reshape(B, H * D)

    out2 = pl.pallas_call(
        body,
        grid=(HG,),
        in_specs=[
            pl.BlockSpec(memory_space=pltpu.MemorySpace.HBM),
            pl.BlockSpec(memory_space=pltpu.MemorySpace.HBM),
            pl.BlockSpec(memory_space=pltpu.MemorySpace.HBM),
            pl.BlockSpec((B, NSLOTS), lambda h: (0, 0)),
            pl.BlockSpec((B, 1), lambda h: (0, 0)),
        ],
        out_specs=pl.BlockSpec((B, H * D), lambda h: (0, 0)),
        out_shape=jax.ShapeDtypeStruct((B, H * D), jnp.float32),
        scratch_shapes=[
            pltpu.VMEM((B, NK), jnp.float32),
            pltpu.VMEM((HG, B, D), jnp.float32),
            pltpu.VMEM((HG, B), jnp.float32),
            pltpu.VMEM((HG, B, D), jnp.float32),
            pltpu.VMEM((HG, B), jnp.float32),
            pltpu.VMEM((HG, B, D), jnp.float32),
            pltpu.VMEM((HG, B, D), jnp.float32),
            pltpu.VMEM((2, B, D), jnp.float32),
            pltpu.VMEM((2, P, BS, D), jnp.float32),
            pltpu.VMEM((2, P, BS, D), jnp.float32),
            pltpu.SemaphoreType.DMA((3, 2, 2)),
            pltpu.SemaphoreType.DMA((3,)),
            pltpu.SemaphoreType.DMA((3,)),
        ],
        compiler_params=pltpu.CompilerParams(
            collective_id=0,
            dimension_semantics=("arbitrary",),
        ),
    )(Q, K, V, bt, lens2)
    return out2.reshape(B, 1, H, D)


# device time: 34596 ns/iter; 4.2673x vs baseline; 1.0336x over previous
import jax
import jax.numpy as jnp
from jax import lax
from jax.experimental import pallas as pl
from jax.experimental.pallas import tpu as pltpu

B, H, D, BS = 32, 16, 128, 32
HG = H // 2
NSLOTS = 256
P = 256
NK = P * BS
PG = 64
SCALE = D ** -0.5


def kernel(Q, K, V, bt, lens):
    lens2 = lens.reshape(B, 1)

    def body(q_hbm, k_hbm, v_hbm, bt_ref, lens_ref, out_ref,
             w_keys, acc_o, acc_l, recv_o, recv_l, xsend, xrecv,
             qbuf, kbuf, vbuf, copy_sems,
             yo_s, yo_r, yl_s, yl_r, xs_s, xs_r):
        h = pl.program_id(0)
        my_x = lax.axis_index("x")
        my_y = lax.axis_index("y")
        ynbr = (my_x, 1 - my_y)
        xnbr = (1 - my_x, my_y)

        def y_rdmas(hh):
            ro = pltpu.make_async_remote_copy(
                src_ref=acc_o.at[hh], dst_ref=recv_o.at[hh],
                send_sem=yo_s.at[hh], recv_sem=yo_r.at[hh],
                device_id=ynbr, device_id_type=pl.DeviceIdType.MESH,
            )
            rl = pltpu.make_async_remote_copy(
                src_ref=acc_l.at[hh], dst_ref=recv_l.at[hh],
                send_sem=yl_s.at[hh], recv_sem=yl_r.at[hh],
                device_id=ynbr, device_id_type=pl.DeviceIdType.MESH,
            )
            return ro, rl

        def x_rdma(wave, col, ncol):
            sl = pl.ds(col, ncol)
            return pltpu.make_async_remote_copy(
                src_ref=xsend.at[:, sl], dst_ref=xrecv.at[:, sl],
                send_sem=xs_s.at[wave], recv_sem=xs_r.at[wave],
                device_id=xnbr, device_id_type=pl.DeviceIdType.MESH,
            )

        def copy_ops(hh, slot):
            ops = [
                pltpu.make_async_copy(
                    q_hbm.at[:, 0, my_x * HG + hh, :], qbuf.at[slot],
                    copy_sems.at[2, slot, 0],
                )
            ]
            for half in range(2):
                sl = pl.ds(half * (P // 2), P // 2)
                ops.append(pltpu.make_async_copy(
                    k_hbm.at[sl, :, my_x * HG + hh, :], kbuf.at[slot, sl],
                    copy_sems.at[0, slot, half],
                ))
                ops.append(pltpu.make_async_copy(
                    v_hbm.at[sl, :, my_x * HG + hh, :], vbuf.at[slot, sl],
                    copy_sems.at[1, slot, half],
                ))
            return ops

        @pl.when(h == 0)
        def _init():
            for op in copy_ops(0, 0):
                op.start()
            barrier = pltpu.get_barrier_semaphore()
            for nbr in (ynbr, xnbr):
                pl.semaphore_signal(
                    barrier, inc=1,
                    device_id=nbr, device_id_type=pl.DeviceIdType.MESH,
                )
            pl.semaphore_wait(barrier, 2)

            slot = lax.broadcasted_iota(jnp.int32, (1, 1, NSLOTS), 2)
            valid = slot < lens_ref[...][:, :, None]
            bt3 = bt_ref[...][:, None, :]
            parts = []
            for g in range(P // PG):
                page_ids = (
                    my_y * P + g * PG
                    + lax.broadcasted_iota(jnp.int32, (1, PG, 1), 1)
                )
                hit = (bt3 == page_ids) & valid
                parts.append(jnp.sum(jnp.where(hit, 1.0, 0.0), axis=2))
            w_page = jnp.concatenate(parts, axis=1)
            w_keys[...] = jnp.broadcast_to(
                w_page[:, :, None], (B, P, BS)
            ).reshape(B, NK)

        @pl.when(h + 1 < HG)
        def _prefetch():
            for op in copy_ops(h + 1, (h + 1) % 2):
                op.start()

        @pl.when(h == HG - 1)
        def _wave_a():
            for hh in range(HG - 1):
                ro, rl = y_rdmas(hh)
                ro.wait()
                rl.wait()
            l_tot = acc_l[0:HG - 1, 0, :] + recv_l[0:HG - 1, 0, :]
            o_tot = acc_o[0:HG - 1] + recv_o[0:HG - 1]
            o_n = o_tot / l_tot[:, :, None]
            xsend[:, 0:(HG - 1) * D] = jnp.transpose(
                o_n, (1, 0, 2)
            ).reshape(B, (HG - 1) * D)
            x_rdma(0, 0, (HG - 1) * D).start()

        for op in copy_ops(h, h % 2):
            op.wait()

        qh = qbuf[h % 2] * SCALE
        kh = kbuf[h % 2].reshape(NK, D)
        vh = vbuf[h % 2].reshape(NK, D)
        s = lax.dot_general(
            qh, kh, (((1,), (1,)), ((), ())),
            preferred_element_type=jnp.float32,
        )
        p = jnp.exp(s) * w_keys[...]
        acc_l[h, 0, :] = jnp.sum(p, axis=1)
        acc_o[h, :, :] = lax.dot_general(
            p, vh, (((1,), (0,)), ((), ())),
            preferred_element_type=jnp.float32,
        )

        ro_h, rl_h = y_rdmas(h)
        ro_h.start()
        rl_h.start()

        @pl.when(h == HG - 1)
        def _wave_b():
            ro, rl = y_rdmas(HG - 1)
            ro.wait()
            rl.wait()
            l7 = acc_l[HG - 1, 0, :] + recv_l[HG - 1, 0, :]
            o7 = (acc_o[HG - 1] + recv_o[HG - 1]) / l7[:, None]
            xsend[:, (HG - 1) * D:HG * D] = o7
            rdma_xb = x_rdma(1, (HG - 1) * D, D)
            rdma_xb.start()
            x_rdma(0, 0, (HG - 1) * D).wait()
            rdma_xb.wait()
            mine = xsend[...]
            theirs = xrecv[...]
            is_x0 = my_x == 0
            g0 = jnp.where(is_x0, mine, theirs)
            g1 = jnp.where(is_x0, theirs, mine)
            out_ref[...] = jnp.concatenate([g0, g1], axis=1)

    out2 = pl.pallas_call(
        body,
        grid=(HG,),
        in_specs=[
            pl.BlockSpec(memory_space=pltpu.MemorySpace.HBM),
            pl.BlockSpec(memory_space=pltpu.MemorySpace.HBM),
            pl.BlockSpec(memory_space=pltpu.MemorySpace.HBM),
            pl.BlockSpec((B, NSLOTS), lambda h: (0, 0)),
            pl.BlockSpec((B, 1), lambda h: (0, 0)),
        ],
        out_specs=pl.BlockSpec((B, H * D), lambda h: (0, 0)),
        out_shape=jax.ShapeDtypeStruct((B, H * D), jnp.float32),
        scratch_shapes=[
            pltpu.VMEM((B, NK), jnp.float32),
            pltpu.VMEM((HG, B, D), jnp.float32),
            pltpu.VMEM((HG, 1, B), jnp.float32),
            pltpu.VMEM((HG, B, D), jnp.float32),
            pltpu.VMEM((HG, 1, B), jnp.float32),
            pltpu.VMEM((B, HG * D), jnp.float32),
            pltpu.VMEM((B, HG * D), jnp.float32),
            pltpu.VMEM((2, B, D), jnp.float32),
            pltpu.VMEM((2, P, BS, D), jnp.float32),
            pltpu.VMEM((2, P, BS, D), jnp.float32),
            pltpu.SemaphoreType.DMA((3, 2, 2)),
            pltpu.SemaphoreType.DMA((HG,)),
            pltpu.SemaphoreType.DMA((HG,)),
            pltpu.SemaphoreType.DMA((HG,)),
            pltpu.SemaphoreType.DMA((HG,)),
            pltpu.SemaphoreType.DMA((2,)),
            pltpu.SemaphoreType.DMA((2,)),
        ],
        compiler_params=pltpu.CompilerParams(
            collective_id=0,
            dimension_semantics=("arbitrary",),
        ),
    )(Q, K, V, bt, lens2)
    return out2.reshape(B, 1, H, D)


# device time: 34192 ns/iter; 4.3178x vs baseline; 1.0118x over previous
import jax
import jax.numpy as jnp
from jax import lax
from jax.experimental import pallas as pl
from jax.experimental.pallas import tpu as pltpu

B, H, D, BS = 32, 16, 128, 32
HG = H // 2
NSLOTS = 256
P = 256
NK = P * BS
PG = 64
SCALE = D ** -0.5


def kernel(Q, K, V, bt, lens):
    lens2 = lens.reshape(B, 1)

    def body(q_hbm, k_hbm, v_hbm, bt_ref, lens_ref, out_ref,
             w_keys, acc_o, acc_l, recv_o, recv_l, xsend, xrecv,
             qbuf, kbuf, vbuf, copy_sems,
             yo_s, yo_r, yl_s, yl_r, xs_s, xs_r):
        h = pl.program_id(0)
        my_x = lax.axis_index("x")
        my_y = lax.axis_index("y")
        ynbr = (my_x, 1 - my_y)
        xnbr = (1 - my_x, my_y)

        def y_rdmas(hh):
            ro = pltpu.make_async_remote_copy(
                src_ref=acc_o.at[hh], dst_ref=recv_o.at[hh],
                send_sem=yo_s.at[hh], recv_sem=yo_r.at[hh],
                device_id=ynbr, device_id_type=pl.DeviceIdType.MESH,
            )
            rl = pltpu.make_async_remote_copy(
                src_ref=acc_l.at[hh], dst_ref=recv_l.at[hh],
                send_sem=yl_s.at[hh], recv_sem=yl_r.at[hh],
                device_id=ynbr, device_id_type=pl.DeviceIdType.MESH,
            )
            return ro, rl

        def x_rdma(wave, col, ncol):
            sl = pl.ds(col, ncol)
            return pltpu.make_async_remote_copy(
                src_ref=xsend.at[:, sl], dst_ref=xrecv.at[:, sl],
                send_sem=xs_s.at[wave], recv_sem=xs_r.at[wave],
                device_id=xnbr, device_id_type=pl.DeviceIdType.MESH,
            )

        def copy_ops(hh, slot):
            ops = [
                pltpu.make_async_copy(
                    q_hbm.at[:, 0, my_x * HG + hh, :], qbuf.at[slot],
                    copy_sems.at[2, slot, 0],
                )
            ]
            for half in range(2):
                sl = pl.ds(half * (P // 2), P // 2)
                ops.append(pltpu.make_async_copy(
                    k_hbm.at[sl, :, my_x * HG + hh, :], kbuf.at[slot, sl],
                    copy_sems.at[0, slot, half],
                ))
                ops.append(pltpu.make_async_copy(
                    v_hbm.at[sl, :, my_x * HG + hh, :], vbuf.at[slot, sl],
                    copy_sems.at[1, slot, half],
                ))
            return ops

        @pl.when(h == 0)
        def _init():
            for op in copy_ops(0, 0):
                op.start()
            barrier = pltpu.get_barrier_semaphore()
            for nbr in (ynbr, xnbr):
                pl.semaphore_signal(
                    barrier, inc=1,
                    device_id=nbr, device_id_type=pl.DeviceIdType.MESH,
                )
            pl.semaphore_wait(barrier, 2)

            slot = lax.broadcasted_iota(jnp.int32, (1, 1, NSLOTS), 2)
            valid = slot < lens_ref[...][:, :, None]
            bt3 = bt_ref[...][:, None, :]
            parts = []
            for g in range(P // PG):
                page_ids = (
                    my_y * P + g * PG
                    + lax.broadcasted_iota(jnp.int32, (1, PG, 1), 1)
                )
                hit = (bt3 == page_ids) & valid
                parts.append(jnp.sum(jnp.where(hit, 1.0, 0.0), axis=2))
            w_page = jnp.concatenate(parts, axis=1)
            w_keys[...] = jnp.broadcast_to(
                w_page[:, :, None], (B, P, BS)
            ).reshape(B, NK).astype(jnp.bfloat16)

        @pl.when(h + 1 < HG)
        def _prefetch():
            for op in copy_ops(h + 1, (h + 1) % 2):
                op.start()

        @pl.when(h == HG - 1)
        def _wave_a():
            for hh in range(HG - 1):
                ro, rl = y_rdmas(hh)
                ro.wait()
                rl.wait()
            l_tot = acc_l[0:HG - 1, 0, :] + recv_l[0:HG - 1, 0, :]
            o_tot = acc_o[0:HG - 1] + recv_o[0:HG - 1]
            o_n = o_tot / l_tot[:, :, None]
            xsend[:, 0:(HG - 1) * D] = jnp.transpose(
                o_n, (1, 0, 2)
            ).reshape(B, (HG - 1) * D)
            x_rdma(0, 0, (HG - 1) * D).start()

        for op in copy_ops(h, h % 2):
            op.wait()

        qh = qbuf[h % 2] * SCALE
        kh = kbuf[h % 2].reshape(NK, D)
        vh = vbuf[h % 2].reshape(NK, D)
        s = lax.dot_general(
            qh, kh, (((1,), (1,)), ((), ())),
            preferred_element_type=jnp.float32,
        )
        p16 = jnp.exp(s.astype(jnp.bfloat16)) * w_keys[...]
        p = p16.astype(jnp.float32)
        acc_l[h, 0, :] = jnp.sum(p16, axis=1, dtype=jnp.float32)
        acc_o[h, :, :] = lax.dot_general(
            p, vh, (((1,), (0,)), ((), ())),
            preferred_element_type=jnp.float32,
        )

        ro_h, rl_h = y_rdmas(h)
        ro_h.start()
        rl_h.start()

        @pl.when(h == HG - 1)
        def _wave_b():
            ro, rl = y_rdmas(HG - 1)
            ro.wait()
            rl.wait()
            l7 = acc_l[HG - 1, 0, :] + recv_l[HG - 1, 0, :]
            o7 = (acc_o[HG - 1] + recv_o[HG - 1]) / l7[:, None]
            xsend[:, (HG - 1) * D:HG * D] = o7
            rdma_xb = x_rdma(1, (HG - 1) * D, D)
            rdma_xb.start()
            x_rdma(0, 0, (HG - 1) * D).wait()
            rdma_xb.wait()
            mine = xsend[...]
            theirs = xrecv[...]
            is_x0 = my_x == 0
            g0 = jnp.where(is_x0, mine, theirs)
            g1 = jnp.where(is_x0, theirs, mine)
            out_ref[...] = jnp.concatenate([g0, g1], axis=1)

    out2 = pl.pallas_call(
        body,
        grid=(HG,),
        in_specs=[
            pl.BlockSpec(memory_space=pltpu.MemorySpace.HBM),
            pl.BlockSpec(memory_space=pltpu.MemorySpace.HBM),
            pl.BlockSpec(memory_space=pltpu.MemorySpace.HBM),
            pl.BlockSpec((B, NSLOTS), lambda h: (0, 0)),
            pl.BlockSpec((B, 1), lambda h: (0, 0)),
        ],
        out_specs=pl.BlockSpec((B, H * D), lambda h: (0, 0)),
        out_shape=jax.ShapeDtypeStruct((B, H * D), jnp.float32),
        scratch_shapes=[
            pltpu.VMEM((B, NK), jnp.bfloat16),
            pltpu.VMEM((HG, B, D), jnp.float32),
            pltpu.VMEM((HG, 1, B), jnp.float32),
            pltpu.VMEM((HG, B, D), jnp.float32),
            pltpu.VMEM((HG, 1, B), jnp.float32),
            pltpu.VMEM((B, HG * D), jnp.float32),
            pltpu.VMEM((B, HG * D), jnp.float32),
            pltpu.VMEM((2, B, D), jnp.float32),
            pltpu.VMEM((2, P, BS, D), jnp.float32),
            pltpu.VMEM((2, P, BS, D), jnp.float32),
            pltpu.SemaphoreType.DMA((3, 2, 2)),
            pltpu.SemaphoreType.DMA((HG,)),
            pltpu.SemaphoreType.DMA((HG,)),
            pltpu.SemaphoreType.DMA((HG,)),
            pltpu.SemaphoreType.DMA((HG,)),
            pltpu.SemaphoreType.DMA((2,)),
            pltpu.SemaphoreType.DMA((2,)),
        ],
        compiler_params=pltpu.CompilerParams(
            collective_id=0,
            dimension_semantics=("arbitrary",),
        ),
    )(Q, K, V, bt, lens2)
    return out2.reshape(B, 1, H, D)
